# baseline (speedup 1.0000x reference)
import functools

import jax
import jax.numpy as jnp
from jax import lax
from jax.experimental import pallas as pl
from jax.experimental.pallas import tpu as pltpu
from jax.experimental.pallas import tpu_sc as plsc

_VOCAB = 1000000
_D = 64
_B = 4096 * 200


@functools.cache
def _build(nw: int, ch: int):
    b_per_w = _B // nw
    n_chunks = b_per_w // ch
    assert b_per_w % ch == 0 and n_chunks % 2 == 0 and n_chunks >= 2
    n_groups = n_chunks // 2
    mesh = plsc.VectorSubcoreMesh(core_axis_name="c", subcore_axis_name="s")

    @functools.partial(
        pl.kernel,
        mesh=mesh,
        out_type=jax.ShapeDtypeStruct((_B, _D), jnp.float32),
        compiler_params=pltpu.CompilerParams(use_tc_tiling_on_sc=False),
        scratch_types=[
            pltpu.VMEM((b_per_w,), jnp.int32),
            pltpu.VMEM((2, ch, _D), jnp.float32),
            pltpu.SemaphoreType.DMA,
            pltpu.SemaphoreType.DMA,
            pltpu.SemaphoreType.DMA,
            pltpu.SemaphoreType.DMA,
        ],
    )
    def k(idx_hbm, table_hbm, out_hbm, idx_v, rows_v, g0, g1, p0, p1):
        nc = 2
        wid = lax.axis_index("s") * nc + lax.axis_index("c")
        base = wid * b_per_w
        pltpu.sync_copy(idx_hbm.at[pl.ds(base, b_per_w)], idx_v)

        gsems = (g0, g1)
        psems = (p0, p1)

        def gather(c, buf):
            pltpu.async_copy(
                table_hbm.at[idx_v.at[pl.ds(c * ch, ch)]],
                rows_v.at[buf],
                gsems[buf],
            )

        def wait_gather(c, buf):
            pltpu.make_async_copy(
                table_hbm.at[idx_v.at[pl.ds(c * ch, ch)]],
                rows_v.at[buf],
                gsems[buf],
            ).wait()

        def put(c, buf):
            pltpu.async_copy(
                rows_v.at[buf],
                out_hbm.at[pl.ds(base + c * ch, ch)],
                psems[buf],
            )

        def wait_put(c, buf):
            pltpu.make_async_copy(
                rows_v.at[buf],
                out_hbm.at[pl.ds(base + c * ch, ch)],
                psems[buf],
            ).wait()

        gather(0, 0)

        def body(g, _):
            c = 2 * g

            @pl.when(g >= 1)
            def _():
                wait_put(c - 1, 1)

            gather(c + 1, 1)
            wait_gather(c, 0)
            put(c, 0)

            @pl.when(g < n_groups - 1)
            def _():
                wait_put(c, 0)
                gather(c + 2, 0)

            wait_gather(c + 1, 1)
            put(c + 1, 1)
            return ()

        lax.fori_loop(0, n_groups, body, (), unroll=False)

        wait_put(n_chunks - 2, 0)
        wait_put(n_chunks - 1, 1)

    return k


def kernel(input, embeddings):
    idx = input.reshape(-1).astype(jnp.int32)
    out = _build(32, 512)(idx, embeddings)
    return out.reshape(input.shape + (_D,))


# speedup vs baseline: 1.0010x; 1.0010x over previous
import functools

import jax
import jax.numpy as jnp
from jax import lax
from jax.experimental import pallas as pl
from jax.experimental.pallas import tpu as pltpu
from jax.experimental.pallas import tpu_sc as plsc

_VOCAB = 1000000
_D = 64
_BATCH = 4096
_HIST = 200


@functools.cache
def _build(nw: int, bb: int):
    rows_per_w = _BATCH // nw
    n_chunks = rows_per_w // bb
    assert rows_per_w % bb == 0 and n_chunks % 2 == 0 and n_chunks >= 2
    n_groups = n_chunks // 2
    mesh = plsc.VectorSubcoreMesh(core_axis_name="c", subcore_axis_name="s")

    @functools.partial(
        pl.kernel,
        mesh=mesh,
        out_type=jax.ShapeDtypeStruct((_BATCH, _HIST, _D), jnp.float32),
        compiler_params=pltpu.CompilerParams(use_tc_tiling_on_sc=False),
        scratch_types=[
            pltpu.VMEM((rows_per_w, _HIST), jnp.int32),
            pltpu.VMEM((2, bb, _HIST, _D), jnp.float32),
            pltpu.SemaphoreType.DMA,
            pltpu.SemaphoreType.DMA,
            pltpu.SemaphoreType.DMA,
            pltpu.SemaphoreType.DMA,
        ],
    )
    def k(idx_hbm, table_hbm, out_hbm, idx_v, rows_v, g0, g1, p0, p1):
        nc = 2
        wid = lax.axis_index("s") * nc + lax.axis_index("c")
        base = wid * rows_per_w
        pltpu.sync_copy(idx_hbm.at[pl.ds(base, rows_per_w)], idx_v)

        gsems = (g0, g1)
        psems = (p0, p1)

        def gather(c, buf):
            for j in range(bb):
                pltpu.async_copy(
                    table_hbm.at[idx_v.at[c * bb + j]],
                    rows_v.at[buf, j],
                    gsems[buf],
                )

        def wait_gather(c, buf):
            for j in range(bb):
                pltpu.make_async_copy(
                    table_hbm.at[idx_v.at[c * bb + j]],
                    rows_v.at[buf, j],
                    gsems[buf],
                ).wait()

        def put(c, buf):
            pltpu.async_copy(
                rows_v.at[buf],
                out_hbm.at[pl.ds(base + c * bb, bb)],
                psems[buf],
            )

        def wait_put(c, buf):
            pltpu.make_async_copy(
                rows_v.at[buf],
                out_hbm.at[pl.ds(base + c * bb, bb)],
                psems[buf],
            ).wait()

        gather(0, 0)

        def body(g, _):
            c = 2 * g

            @pl.when(g >= 1)
            def _():
                wait_put(c - 1, 1)

            gather(c + 1, 1)
            wait_gather(c, 0)
            put(c, 0)

            @pl.when(g < n_groups - 1)
            def _():
                wait_put(c, 0)
                gather(c + 2, 0)

            wait_gather(c + 1, 1)
            put(c + 1, 1)
            return ()

        lax.fori_loop(0, n_groups, body, (), unroll=False)

        wait_put(n_chunks - 2, 0)
        wait_put(n_chunks - 1, 1)

    return k


def kernel(input, embeddings):
    idx = input.astype(jnp.int32)
    return _build(32, 4)(idx, embeddings)
